# baseline (speedup 1.0000x reference)
import functools

import jax
import jax.numpy as jnp
from jax import lax
from jax.experimental import pallas as pl
from jax.experimental.pallas import tpu as pltpu
from jax.experimental.pallas import tpu_sc as plsc

N_NODES = 10000
N_PAD = 10240
D = 128
N_EDGES = 320000
NC = 2
NS = 16
NW = NC * NS
CH = 128
NCHUNK = N_EDGES // CH
NFULL = NCHUNK // NW
NEXTRA = NCHUNK - NFULL * NW
ROWS_PER_TILE = N_PAD // NS
LANES = 16
NB = 2
T_MAX = 80
MM_GRID = 10


def _mm_body(x_ref, w_ref, o_ref):
    o_ref[...] = jnp.dot(x_ref[...], w_ref[...],
                         preferred_element_type=jnp.float32)


def _prep_body(ei_ref, o_src, o_dst):
    o_src[...] = ei_ref[1, :]
    o_dst[...] = ei_ref[0, :]


def _combine_body(p_ref, o_ref):
    o_ref[...] = jnp.maximum(p_ref[0] + p_ref[1], 0.0)


def _bcast_lane(vec, lane):
    idx = jnp.full((LANES, 1), lane, jnp.int32)
    dnums = lax.GatherDimensionNumbers(
        offset_dims=(), collapsed_slice_dims=(0,), start_index_map=(0,))
    return lax.gather(vec, idx, dnums, (1,),
                      mode=lax.GatherScatterMode.PROMISE_IN_BOUNDS)


def _sc_scatter_body(xw, src1d, dst1d, evals, out,
                     sbuf0, sbuf1, ubuf0, ubuf1, vbuf0, vbuf1,
                     rows0, rows1, dbuf0, dbuf1, acc,
                     isem0, isem1, gsem0, gsem1, ssem0, ssem1):
    c = lax.axis_index("c")
    s = lax.axis_index("s")
    w = c * NS + s
    n_w = NFULL + jnp.where(w < NEXTRA, 1, 0)
    sbuf = (sbuf0, sbuf1)
    ubuf = (ubuf0, ubuf1)
    vbuf = (vbuf0, vbuf1)
    rows = (rows0, rows1)
    dbuf = (dbuf0, dbuf1)
    isem = (isem0, isem1)
    gsem = (gsem0, gsem1)
    ssem = (ssem0, ssem1)

    def issue_idx(t, p):
        base = (w + t * NW) * CH
        pltpu.async_copy(src1d.at[pl.ds(base, CH)], sbuf[p], isem[p])
        pltpu.async_copy(dst1d.at[pl.ds(base, CH)], ubuf[p], isem[p])
        pltpu.async_copy(evals.at[pl.ds(base, CH)], vbuf[p], isem[p])

    def wait_idx(p):
        pltpu.make_async_copy(src1d.at[pl.ds(0, CH)], sbuf[p],
                              isem[p]).wait()
        pltpu.make_async_copy(dst1d.at[pl.ds(0, CH)], ubuf[p],
                              isem[p]).wait()
        pltpu.make_async_copy(evals.at[pl.ds(0, CH)], vbuf[p],
                              isem[p]).wait()

    def zrow(r, carry):
        for j in range(D // LANES):
            rows0[r, pl.ds(j * LANES, LANES)] = jnp.zeros((LANES,),
                                                          jnp.float32)
        return carry
    lax.fori_loop(0, CH, zrow, 0)
    for k in range(ROWS_PER_TILE // CH):
        pltpu.sync_copy(rows0, acc.at[pl.ds(s * ROWS_PER_TILE + k * CH, CH)])

    issue_idx(0, 0)
    issue_idx(1, 1)
    wait_idx(0)
    pltpu.async_copy(xw.at[sbuf0], rows0, gsem0)

    plsc.subcore_barrier()

    def outer_body(i, carry):
        for p in range(NB):
            t = i * NB + p
            q = 1 - p

            @pl.when(t + 1 < n_w)
            def _():
                wait_idx(q)

            @pl.when(jnp.logical_and(t >= 1, t - 1 < n_w))
            def _():
                pltpu.make_async_copy(
                    rows[q], acc.at[dbuf[q].at[0]], ssem[q]).wait()

            @pl.when(t + 1 < n_w)
            def _():
                pltpu.async_copy(xw.at[sbuf[q]], rows[q], gsem[q])

            @pl.when(t < n_w)
            def _():
                pltpu.make_async_copy(xw.at[sbuf[p]], rows[p],
                                      gsem[p]).wait()

                def group_body(g, gcarry):
                    vals16 = vbuf[p][pl.ds(g * LANES, LANES)]
                    for l in range(LANES):
                        vv = _bcast_lane(vals16, l)
                        e = g * LANES + l
                        for j in range(D // LANES):
                            sl = pl.ds(j * LANES, LANES)
                            rows[p][e, sl] = rows[p][e, sl] * vv
                    return gcarry
                lax.fori_loop(0, CH // LANES, group_body, 0)

                for j in range(CH // LANES):
                    sl = pl.ds(j * LANES, LANES)
                    dbuf[p][0, sl] = ubuf[p][sl]
                pltpu.async_copy(rows[p], acc.at[dbuf[p].at[0]], ssem[p],
                                 add=True)

            @pl.when(t + NB < n_w)
            def _():
                issue_idx(t + NB, p)
        return carry
    lax.fori_loop(0, T_MAX // NB, outer_body, 0)

    plsc.subcore_barrier()
    pltpu.sync_copy(acc.at[pl.ds(s * ROWS_PER_TILE, ROWS_PER_TILE)],
                    out.at[c, pl.ds(s * ROWS_PER_TILE, ROWS_PER_TILE)])


_sc_scatter = functools.partial(
    pl.kernel,
    mesh=plsc.VectorSubcoreMesh(core_axis_name="c", subcore_axis_name="s"),
    out_type=jax.ShapeDtypeStruct((NC, N_PAD, D), jnp.float32),
    scratch_types=(
        [pltpu.VMEM((CH,), jnp.int32) for _ in range(NB)]
        + [pltpu.VMEM((CH,), jnp.int32) for _ in range(NB)]
        + [pltpu.VMEM((CH,), jnp.float32) for _ in range(NB)]
        + [pltpu.VMEM((CH, D), jnp.float32) for _ in range(NB)]
        + [pltpu.VMEM((1, CH), jnp.int32) for _ in range(NB)]
        + [pltpu.VMEM_SHARED((N_PAD, D), jnp.float32)]
        + [pltpu.SemaphoreType.DMA for _ in range(3 * NB)]
    ),
)(_sc_scatter_body)


def kernel(x, edge_index, edge_vals, W):
    xw = pl.pallas_call(
        _mm_body,
        grid=(MM_GRID,),
        in_specs=[
            pl.BlockSpec((N_NODES // MM_GRID, D), lambda i: (i, 0)),
            pl.BlockSpec((D, D), lambda i: (0, 0)),
        ],
        out_specs=pl.BlockSpec((N_NODES // MM_GRID, D), lambda i: (i, 0)),
        out_shape=jax.ShapeDtypeStruct((N_NODES, D), jnp.float32),
    )(x, W)

    PB = 32768
    NPB = 10
    src1d, dst1d = pl.pallas_call(
        _prep_body,
        grid=(NPB,),
        in_specs=[pl.BlockSpec((2, PB), lambda i: (0, i))],
        out_specs=[
            pl.BlockSpec((PB,), lambda i: (i,)),
            pl.BlockSpec((PB,), lambda i: (i,)),
        ],
        out_shape=[
            jax.ShapeDtypeStruct((PB * NPB,), jnp.int32),
            jax.ShapeDtypeStruct((PB * NPB,), jnp.int32),
        ],
    )(edge_index.astype(jnp.int32))

    partials = _sc_scatter(xw, src1d, dst1d, edge_vals)

    out = pl.pallas_call(
        _combine_body,
        grid=(10,),
        in_specs=[pl.BlockSpec((NC, N_NODES // 10, D), lambda i: (0, i, 0))],
        out_specs=pl.BlockSpec((N_NODES // 10, D), lambda i: (i, 0)),
        out_shape=jax.ShapeDtypeStruct((N_NODES, D), jnp.float32),
    )(partials)
    return out


# speedup vs baseline: 1.1089x; 1.1089x over previous
import functools

import jax
import jax.numpy as jnp
from jax import lax
from jax.experimental import pallas as pl
from jax.experimental.pallas import tpu as pltpu
from jax.experimental.pallas import tpu_sc as plsc

N_NODES = 10000
N_PAD = 10240
D = 128
N_EDGES = 320000
NC = 2
NS = 16
NW = NC * NS
CH = 64
NCHUNK = N_EDGES // CH
NFULL = NCHUNK // NW
NEXTRA = NCHUNK - NFULL * NW
ROWS_PER_TILE = N_PAD // NS
LANES = 16
NB = 4
T_MAX = 160
MM_GRID = 10


def _mm_body(x_ref, w_ref, o_ref):
    o_ref[...] = jnp.dot(x_ref[...], w_ref[...],
                         preferred_element_type=jnp.float32)


def _prep_body(ei_ref, o_src, o_dst):
    o_src[...] = ei_ref[1, :]
    o_dst[...] = ei_ref[0, :]


def _combine_body(p_ref, o_ref):
    o_ref[...] = jnp.maximum(p_ref[0] + p_ref[1], 0.0)


def _bcast_lane(vec, lane):
    idx = jnp.full((LANES, 1), lane, jnp.int32)
    dnums = lax.GatherDimensionNumbers(
        offset_dims=(), collapsed_slice_dims=(0,), start_index_map=(0,))
    return lax.gather(vec, idx, dnums, (1,),
                      mode=lax.GatherScatterMode.PROMISE_IN_BOUNDS)


def _sc_scatter_body(xw, src1d, dst1d, evals, out,
                     sbuf0, sbuf1, sbuf2, sbuf3,
                     ubuf0, ubuf1, ubuf2, ubuf3,
                     vbuf0, vbuf1, vbuf2, vbuf3,
                     rows0, rows1, rows2, rows3,
                     dbuf0, dbuf1, dbuf2, dbuf3, acc,
                     isem0, isem1, isem2, isem3,
                     gsem0, gsem1, gsem2, gsem3,
                     ssem0, ssem1, ssem2, ssem3):
    c = lax.axis_index("c")
    s = lax.axis_index("s")
    w = c * NS + s
    n_w = NFULL + jnp.where(w < NEXTRA, 1, 0)
    sbuf = (sbuf0, sbuf1, sbuf2, sbuf3)
    ubuf = (ubuf0, ubuf1, ubuf2, ubuf3)
    vbuf = (vbuf0, vbuf1, vbuf2, vbuf3)
    rows = (rows0, rows1, rows2, rows3)
    dbuf = (dbuf0, dbuf1, dbuf2, dbuf3)
    isem = (isem0, isem1, isem2, isem3)
    gsem = (gsem0, gsem1, gsem2, gsem3)
    ssem = (ssem0, ssem1, ssem2, ssem3)

    def issue_idx(t, p):
        base = (w + t * NW) * CH
        pltpu.async_copy(src1d.at[pl.ds(base, CH)], sbuf[p], isem[p])
        pltpu.async_copy(dst1d.at[pl.ds(base, CH)], ubuf[p], isem[p])
        pltpu.async_copy(evals.at[pl.ds(base, CH)], vbuf[p], isem[p])

    def wait_idx(p):
        pltpu.make_async_copy(src1d.at[pl.ds(0, CH)], sbuf[p],
                              isem[p]).wait()
        pltpu.make_async_copy(dst1d.at[pl.ds(0, CH)], ubuf[p],
                              isem[p]).wait()
        pltpu.make_async_copy(evals.at[pl.ds(0, CH)], vbuf[p],
                              isem[p]).wait()

    def zrow(r, carry):
        for j in range(D // LANES):
            rows0[r, pl.ds(j * LANES, LANES)] = jnp.zeros((LANES,),
                                                          jnp.float32)
        return carry
    lax.fori_loop(0, CH, zrow, 0)
    for k in range(ROWS_PER_TILE // CH):
        pltpu.sync_copy(rows0, acc.at[pl.ds(s * ROWS_PER_TILE + k * CH, CH)])

    for k in range(NB):
        issue_idx(k, k)
    wait_idx(0)
    pltpu.async_copy(xw.at[sbuf0], rows0, gsem0)
    wait_idx(1)
    pltpu.async_copy(xw.at[sbuf1], rows1, gsem1)

    plsc.subcore_barrier()

    def outer_body(i, carry):
        for p in range(NB):
            t = i * NB + p
            qq = (p + 2) % NB

            @pl.when(jnp.logical_and(t >= 2, t - 2 < n_w))
            def _():
                pltpu.make_async_copy(
                    rows[qq], acc.at[dbuf[qq].at[0]], ssem[qq]).wait()

            @pl.when(t + 2 < n_w)
            def _():
                wait_idx(qq)
                pltpu.async_copy(xw.at[sbuf[qq]], rows[qq], gsem[qq])

            @pl.when(t < n_w)
            def _():
                pltpu.make_async_copy(xw.at[sbuf[p]], rows[p],
                                      gsem[p]).wait()

                def group_body(g, gcarry):
                    vals16 = vbuf[p][pl.ds(g * LANES, LANES)]
                    for l in range(LANES):
                        vv = _bcast_lane(vals16, l)
                        e = g * LANES + l
                        for j in range(D // LANES):
                            sl = pl.ds(j * LANES, LANES)
                            rows[p][e, sl] = rows[p][e, sl] * vv
                    return gcarry
                lax.fori_loop(0, CH // LANES, group_body, 0)

                for j in range(CH // LANES):
                    sl = pl.ds(j * LANES, LANES)
                    dbuf[p][0, sl] = ubuf[p][sl]
                pltpu.async_copy(rows[p], acc.at[dbuf[p].at[0]], ssem[p],
                                 add=True)

            @pl.when(t + NB < n_w)
            def _():
                issue_idx(t + NB, p)
        return carry
    lax.fori_loop(0, T_MAX // NB, outer_body, 0)

    plsc.subcore_barrier()
    pltpu.sync_copy(acc.at[pl.ds(s * ROWS_PER_TILE, ROWS_PER_TILE)],
                    out.at[c, pl.ds(s * ROWS_PER_TILE, ROWS_PER_TILE)])


_sc_scatter = functools.partial(
    pl.kernel,
    mesh=plsc.VectorSubcoreMesh(core_axis_name="c", subcore_axis_name="s"),
    out_type=jax.ShapeDtypeStruct((NC, N_PAD, D), jnp.float32),
    scratch_types=(
        [pltpu.VMEM((CH,), jnp.int32) for _ in range(NB)]
        + [pltpu.VMEM((CH,), jnp.int32) for _ in range(NB)]
        + [pltpu.VMEM((CH,), jnp.float32) for _ in range(NB)]
        + [pltpu.VMEM((CH, D), jnp.float32) for _ in range(NB)]
        + [pltpu.VMEM((1, CH), jnp.int32) for _ in range(NB)]
        + [pltpu.VMEM_SHARED((N_PAD, D), jnp.float32)]
        + [pltpu.SemaphoreType.DMA for _ in range(3 * NB)]
    ),
)(_sc_scatter_body)


def kernel(x, edge_index, edge_vals, W):
    xw = pl.pallas_call(
        _mm_body,
        grid=(MM_GRID,),
        in_specs=[
            pl.BlockSpec((N_NODES // MM_GRID, D), lambda i: (i, 0)),
            pl.BlockSpec((D, D), lambda i: (0, 0)),
        ],
        out_specs=pl.BlockSpec((N_NODES // MM_GRID, D), lambda i: (i, 0)),
        out_shape=jax.ShapeDtypeStruct((N_NODES, D), jnp.float32),
    )(x, W)

    PB = 32768
    NPB = 10
    src1d, dst1d = pl.pallas_call(
        _prep_body,
        grid=(NPB,),
        in_specs=[pl.BlockSpec((2, PB), lambda i: (0, i))],
        out_specs=[
            pl.BlockSpec((PB,), lambda i: (i,)),
            pl.BlockSpec((PB,), lambda i: (i,)),
        ],
        out_shape=[
            jax.ShapeDtypeStruct((PB * NPB,), jnp.int32),
            jax.ShapeDtypeStruct((PB * NPB,), jnp.int32),
        ],
    )(edge_index.astype(jnp.int32))

    partials = _sc_scatter(xw, src1d, dst1d, edge_vals)

    out = pl.pallas_call(
        _combine_body,
        grid=(10,),
        in_specs=[pl.BlockSpec((NC, N_NODES // 10, D), lambda i: (0, i, 0))],
        out_specs=pl.BlockSpec((N_NODES // 10, D), lambda i: (i, 0)),
        out_shape=jax.ShapeDtypeStruct((N_NODES, D), jnp.float32),
    )(partials)
    return out


# speedup vs baseline: 1.1543x; 1.0409x over previous
import functools

import jax
import jax.numpy as jnp
from jax import lax
from jax.experimental import pallas as pl
from jax.experimental.pallas import tpu as pltpu
from jax.experimental.pallas import tpu_sc as plsc

N_NODES = 10000
N_PAD = 10240
D = 128
N_EDGES = 320000
NC = 2
NS = 16
NW = NC * NS
CH = 64
NCHUNK = N_EDGES // CH
NFULL = NCHUNK // NW
NEXTRA = NCHUNK - NFULL * NW
ROWS_PER_TILE = N_PAD // NS
LANES = 16
NB = 5
GDEPTH = 3
T_MAX = 160
MM_GRID = 10
PB = 32768


def _mm_prep_body(x_ref, w_ref, ei_ref, o_xw, o_src, o_dst):
    o_xw[...] = jnp.dot(x_ref[...], w_ref[...],
                        preferred_element_type=jnp.float32)
    o_src[...] = ei_ref[1, :]
    o_dst[...] = ei_ref[0, :]


def _combine_body(p_ref, o_ref):
    o_ref[...] = jnp.maximum(p_ref[0] + p_ref[1], 0.0)


def _bcast_lane(vec, lane):
    idx = jnp.full((LANES, 1), lane, jnp.int32)
    dnums = lax.GatherDimensionNumbers(
        offset_dims=(), collapsed_slice_dims=(0,), start_index_map=(0,))
    return lax.gather(vec, idx, dnums, (1,),
                      mode=lax.GatherScatterMode.PROMISE_IN_BOUNDS)


def _sc_scatter_body(xw, src1d, dst1d, evals, out, *refs):
    sbuf = refs[0:NB]
    ubuf = refs[NB:2 * NB]
    vbuf = refs[2 * NB:3 * NB]
    rows = refs[3 * NB:4 * NB]
    dbuf = refs[4 * NB:5 * NB]
    acc = refs[5 * NB]
    isem = refs[5 * NB + 1:5 * NB + 1 + NB]
    gsem = refs[5 * NB + 1 + NB:5 * NB + 1 + 2 * NB]
    ssem = refs[5 * NB + 1 + 2 * NB:5 * NB + 1 + 3 * NB]

    c = lax.axis_index("c")
    s = lax.axis_index("s")
    w = c * NS + s
    n_w = NFULL + jnp.where(w < NEXTRA, 1, 0)

    def issue_idx(t, p):
        base = (w + t * NW) * CH
        pltpu.async_copy(src1d.at[pl.ds(base, CH)], sbuf[p], isem[p])
        pltpu.async_copy(dst1d.at[pl.ds(base, CH)], ubuf[p], isem[p])
        pltpu.async_copy(evals.at[pl.ds(base, CH)], vbuf[p], isem[p])

    def wait_idx(p):
        pltpu.make_async_copy(src1d.at[pl.ds(0, CH)], sbuf[p],
                              isem[p]).wait()
        pltpu.make_async_copy(dst1d.at[pl.ds(0, CH)], ubuf[p],
                              isem[p]).wait()
        pltpu.make_async_copy(evals.at[pl.ds(0, CH)], vbuf[p],
                              isem[p]).wait()

    def zrow(r, carry):
        for j in range(D // LANES):
            rows[0][r, pl.ds(j * LANES, LANES)] = jnp.zeros((LANES,),
                                                            jnp.float32)
        return carry
    lax.fori_loop(0, CH, zrow, 0)
    for k in range(ROWS_PER_TILE // CH):
        pltpu.sync_copy(rows[0],
                        acc.at[pl.ds(s * ROWS_PER_TILE + k * CH, CH)])

    for k in range(NB):
        issue_idx(k, k)
    for k in range(GDEPTH):
        wait_idx(k)
        pltpu.async_copy(xw.at[sbuf[k]], rows[k], gsem[k])

    plsc.subcore_barrier()

    def outer_body(i, carry):
        for p in range(NB):
            t = i * NB + p
            qq = (p + GDEPTH) % NB

            @pl.when(jnp.logical_and(t >= 2, t - 2 < n_w))
            def _():
                sp = (p + NB - 2) % NB
                pltpu.make_async_copy(
                    rows[sp], acc.at[dbuf[sp].at[0]], ssem[sp]).wait()

            @pl.when(t + GDEPTH < n_w)
            def _():
                wait_idx(qq)
                pltpu.async_copy(xw.at[sbuf[qq]], rows[qq], gsem[qq])

            @pl.when(t < n_w)
            def _():
                pltpu.make_async_copy(xw.at[sbuf[p]], rows[p],
                                      gsem[p]).wait()

                def group_body(g, gcarry):
                    vals16 = vbuf[p][pl.ds(g * LANES, LANES)]
                    for l in range(LANES):
                        vv = _bcast_lane(vals16, l)
                        e = g * LANES + l
                        for j in range(D // LANES):
                            sl = pl.ds(j * LANES, LANES)
                            rows[p][e, sl] = rows[p][e, sl] * vv
                    return gcarry
                lax.fori_loop(0, CH // LANES, group_body, 0)

                for j in range(CH // LANES):
                    sl = pl.ds(j * LANES, LANES)
                    dbuf[p][0, sl] = ubuf[p][sl]
                pltpu.async_copy(rows[p], acc.at[dbuf[p].at[0]], ssem[p],
                                 add=True)

            @pl.when(t + NB < n_w)
            def _():
                issue_idx(t + NB, p)
        return carry
    lax.fori_loop(0, T_MAX // NB, outer_body, 0)

    plsc.subcore_barrier()
    pltpu.sync_copy(acc.at[pl.ds(s * ROWS_PER_TILE, ROWS_PER_TILE)],
                    out.at[c, pl.ds(s * ROWS_PER_TILE, ROWS_PER_TILE)])


_sc_scatter = functools.partial(
    pl.kernel,
    mesh=plsc.VectorSubcoreMesh(core_axis_name="c", subcore_axis_name="s"),
    out_type=jax.ShapeDtypeStruct((NC, N_PAD, D), jnp.float32),
    scratch_types=(
        [pltpu.VMEM((CH,), jnp.int32) for _ in range(NB)]
        + [pltpu.VMEM((CH,), jnp.int32) for _ in range(NB)]
        + [pltpu.VMEM((CH,), jnp.float32) for _ in range(NB)]
        + [pltpu.VMEM((CH, D), jnp.float32) for _ in range(NB)]
        + [pltpu.VMEM((1, CH), jnp.int32) for _ in range(NB)]
        + [pltpu.VMEM_SHARED((N_PAD, D), jnp.float32)]
        + [pltpu.SemaphoreType.DMA for _ in range(3 * NB)]
    ),
)(_sc_scatter_body)


def kernel(x, edge_index, edge_vals, W):
    xw, src1d, dst1d = pl.pallas_call(
        _mm_prep_body,
        grid=(MM_GRID,),
        in_specs=[
            pl.BlockSpec((N_NODES // MM_GRID, D), lambda i: (i, 0)),
            pl.BlockSpec((D, D), lambda i: (0, 0)),
            pl.BlockSpec((2, PB), lambda i: (0, i)),
        ],
        out_specs=[
            pl.BlockSpec((N_NODES // MM_GRID, D), lambda i: (i, 0)),
            pl.BlockSpec((PB,), lambda i: (i,)),
            pl.BlockSpec((PB,), lambda i: (i,)),
        ],
        out_shape=[
            jax.ShapeDtypeStruct((N_NODES, D), jnp.float32),
            jax.ShapeDtypeStruct((PB * MM_GRID,), jnp.int32),
            jax.ShapeDtypeStruct((PB * MM_GRID,), jnp.int32),
        ],
    )(x, W, edge_index.astype(jnp.int32))

    partials = _sc_scatter(xw, src1d, dst1d, edge_vals)

    out = pl.pallas_call(
        _combine_body,
        grid=(10,),
        in_specs=[pl.BlockSpec((NC, N_NODES // 10, D), lambda i: (0, i, 0))],
        out_specs=pl.BlockSpec((N_NODES // 10, D), lambda i: (i, 0)),
        out_shape=jax.ShapeDtypeStruct((N_NODES, D), jnp.float32),
    )(partials)
    return out
